# Initial kernel scaffold; baseline (speedup 1.0000x reference)
import functools
import math

import jax
import jax.numpy as jnp
from jax import lax
from jax.experimental import pallas as pl
from jax.experimental.pallas import tpu as pltpu

B, S, D, H, C, NB, DFF, V = 2, 4096, 1024, 8, 128, 32, 4096, 256
DH = D // H
NC = S // C
BH = B * H
NBK = 2 * NB


_SB1 = 512


def _stage1_body(x_ref, g_ref, b_ref, wqk_ref, wv_ref, rot_ref,
                 qk_ref, v_ref, bk_ref):
    x = x_ref[0]
    mu = jnp.mean(x, axis=-1, keepdims=True)
    var = jnp.mean((x - mu) * (x - mu), axis=-1, keepdims=True)
    h = (x - mu) * lax.rsqrt(var + 1e-6) * g_ref[...] + b_ref[...]
    qk = jnp.dot(h, wqk_ref[...], preferred_element_type=jnp.float32)
    v = jnp.dot(h, wv_ref[...], preferred_element_type=jnp.float32)
    rx = jnp.dot(qk, rot_ref[...], preferred_element_type=jnp.float32)
    cols = []
    for hh in range(H):
        qk_ref[0, hh] = qk[:, hh * DH:(hh + 1) * DH]
        v_ref[0, hh] = v[:, hh * DH:(hh + 1) * DH]
        r = rx[:, hh * NBK:(hh + 1) * NBK]
        m = jnp.max(r, axis=-1, keepdims=True)
        ii = lax.broadcasted_iota(jnp.int32, r.shape, 1)
        idx = jnp.min(jnp.where(r == m, ii, NBK), axis=-1)
        cols.append(idx[:, None])
    bk_ref[:, 0, :] = jnp.concatenate(cols, axis=1)


def _stage1(x, ln1_g, ln1_b, Wqk, Wv, rot_bd, interpret=False):
    grid = (B, S // _SB1)
    return pl.pallas_call(
        _stage1_body,
        grid=grid,
        in_specs=[
            pl.BlockSpec((1, _SB1, D), lambda b, i: (b, i, 0)),
            pl.BlockSpec((D,), lambda b, i: (0,)),
            pl.BlockSpec((D,), lambda b, i: (0,)),
            pl.BlockSpec((D, D), lambda b, i: (0, 0)),
            pl.BlockSpec((D, D), lambda b, i: (0, 0)),
            pl.BlockSpec((D, H * NBK), lambda b, i: (0, 0)),
        ],
        out_specs=[
            pl.BlockSpec((1, H, _SB1, DH), lambda b, i: (b, 0, i, 0)),
            pl.BlockSpec((1, H, _SB1, DH), lambda b, i: (b, 0, i, 0)),
            pl.BlockSpec((_SB1, 1, H), lambda b, i: (i, b, 0)),
        ],
        out_shape=[
            jax.ShapeDtypeStruct((B, H, S, DH), jnp.float32),
            jax.ShapeDtypeStruct((B, H, S, DH), jnp.float32),
            jax.ShapeDtypeStruct((S, B, H), jnp.int32),
        ],
        interpret=interpret,
    )(x, ln1_g, ln1_b, Wqk, Wv, rot_bd)


def _attn_body(q_ref, kp_ref, kc_ref, vp_ref, vc_ref, tp_ref, tc_ref, o_ref):
    p = pl.program_id(0)
    q = q_ref[0]
    k2 = jnp.concatenate([kp_ref[0], kc_ref[0]], axis=0)
    nrm = jnp.sqrt(jnp.sum(k2 * k2, axis=-1, keepdims=True))
    k2 = k2 / (nrm + 1e-6)
    s = lax.dot_general(q, k2, (((1,), (1,)), ((), ())),
                        preferred_element_type=jnp.float32)
    s = s * (1.0 / math.sqrt(DH))
    base = p * S
    tq = (tc_ref[0, 0, 0] - base).astype(jnp.float32)
    tk = (jnp.concatenate([tp_ref[0, 0, 0], tc_ref[0, 0, 0]], axis=1)
          - base).astype(jnp.float32)
    tqc = jnp.transpose(tq)
    s = jnp.where(tqc < tk, -1e9, s)
    s = jnp.where(tqc == tk, -1e5, s)
    m = jnp.max(s, axis=-1, keepdims=True)
    e = jnp.exp(s - m)
    a = e / jnp.sum(e, axis=-1, keepdims=True)
    v2 = jnp.concatenate([vp_ref[0], vc_ref[0]], axis=0)
    o_ref[0] = jnp.dot(a, v2, preferred_element_type=jnp.float32)


def _attention(qks, vs, ts, interpret=False):
    grid = (BH, NC)
    prev = lambda n: (n + NC - 1) % NC
    return pl.pallas_call(
        _attn_body,
        grid=grid,
        in_specs=[
            pl.BlockSpec((1, C, DH), lambda p, n: (p, n, 0)),
            pl.BlockSpec((1, C, DH), lambda p, n: (p, prev(n), 0)),
            pl.BlockSpec((1, C, DH), lambda p, n: (p, n, 0)),
            pl.BlockSpec((1, C, DH), lambda p, n: (p, prev(n), 0)),
            pl.BlockSpec((1, C, DH), lambda p, n: (p, n, 0)),
            pl.BlockSpec((1, 1, 1, C), lambda p, n: (p, prev(n), 0, 0)),
            pl.BlockSpec((1, 1, 1, C), lambda p, n: (p, n, 0, 0)),
        ],
        out_specs=pl.BlockSpec((1, C, DH), lambda p, n: (p, n, 0)),
        out_shape=jax.ShapeDtypeStruct((BH, S, DH), jnp.float32),
        interpret=interpret,
    )(qks, qks, qks, vs, vs, ts, ts)


_SB6 = 256
_KCH = 4


def _stage6_body(x_ref, o_ref, wo_ref, g2_ref, b2_ref, w1_ref, bb1_ref,
                 w2_ref, bb2_ref, wout_ref, bout_ref, out_ref):
    R = 2 * _SB6
    x2 = x_ref[...].reshape(R, D)
    o2 = o_ref[...].reshape(R, D)
    y1 = x2 + jnp.dot(o2, wo_ref[...], preferred_element_type=jnp.float32)
    mu = jnp.mean(y1, axis=-1, keepdims=True)
    var = jnp.mean((y1 - mu) * (y1 - mu), axis=-1, keepdims=True)
    h2 = (y1 - mu) * lax.rsqrt(var + 1e-6) * g2_ref[...] + b2_ref[...]
    kc = DFF // _KCH
    acc = jnp.zeros((R, D), jnp.float32)
    for j in range(_KCH):
        hj = jnp.dot(h2, w1_ref[:, j * kc:(j + 1) * kc],
                     preferred_element_type=jnp.float32)
        hj = jnp.maximum(hj + bb1_ref[j * kc:(j + 1) * kc], 0.0)
        acc = acc + jnp.dot(hj, w2_ref[j * kc:(j + 1) * kc, :],
                            preferred_element_type=jnp.float32)
    y2 = x2 + acc + bb2_ref[...]
    lg = (jnp.dot(y1, wout_ref[:D], preferred_element_type=jnp.float32)
          + jnp.dot(y2, wout_ref[D:], preferred_element_type=jnp.float32)
          + bout_ref[...])
    l0 = lg[:_SB6]
    l1 = lg[_SB6:]
    m = jnp.maximum(l0, l1)
    e0 = jnp.exp(l0 - m)
    e1 = jnp.exp(l1 - m)
    ssum = e0 + e1
    out_ref[0] = e0 / ssum
    out_ref[1] = e1 / ssum


def _stage6(x, o_t, Wo, ln2_g, ln2_b, W1, b1, W2, b2, Wout, bout,
            interpret=False):
    grid = (S // _SB6,)
    return pl.pallas_call(
        _stage6_body,
        grid=grid,
        in_specs=[
            pl.BlockSpec((B, _SB6, D), lambda i: (0, i, 0)),
            pl.BlockSpec((B, _SB6, D), lambda i: (0, i, 0)),
            pl.BlockSpec((D, D), lambda i: (0, 0)),
            pl.BlockSpec((D,), lambda i: (0,)),
            pl.BlockSpec((D,), lambda i: (0,)),
            pl.BlockSpec((D, DFF), lambda i: (0, 0)),
            pl.BlockSpec((DFF,), lambda i: (0,)),
            pl.BlockSpec((DFF, D), lambda i: (0, 0)),
            pl.BlockSpec((D,), lambda i: (0,)),
            pl.BlockSpec((2 * D, V), lambda i: (0, 0)),
            pl.BlockSpec((V,), lambda i: (0,)),
        ],
        out_specs=pl.BlockSpec((B, _SB6, V), lambda i: (0, i, 0)),
        out_shape=jax.ShapeDtypeStruct((B, S, V), jnp.float32),
        interpret=interpret,
    )(x, o_t, Wo, ln2_g, ln2_b, W1, b1, W2, b2, Wout, bout)


def _sort_pairs(bk_sbh):
    bk = bk_sbh.reshape(S, BH).T
    t = jnp.arange(S, dtype=jnp.int32)
    sticker = bk * S + t[None, :]
    s_idx = jnp.argsort(sticker, axis=-1).astype(jnp.int32)
    u_idx = jnp.argsort(s_idx, axis=-1).astype(jnp.int32)
    base = (jnp.arange(BH, dtype=jnp.int32) * S)[:, None]
    return s_idx + base, u_idx + base


def _gather_rows(table, idx_flat):
    return jnp.take(table, idx_flat, axis=0)


def _pipeline(x, rot, Wqk, Wv, Wo, ln1_g, ln1_b, W1, b1, W2, b2,
              ln2_g, ln2_b, Wout, bout, interpret=False):
    rotc = jnp.concatenate([rot, -rot], axis=-1)
    rot_bd = jnp.zeros((D, H * NBK), jnp.float32)
    for hh in range(H):
        rot_bd = rot_bd.at[hh * DH:(hh + 1) * DH,
                           hh * NBK:(hh + 1) * NBK].set(rotc[hh])

    qk, v, bk_sbh = _stage1(x, ln1_g, ln1_b, Wqk, Wv, rot_bd,
                            interpret=interpret)

    s_idx_g, u_idx_g = _sort_pairs(bk_sbh)

    qk2 = qk.reshape(BH * S, DH)
    v2 = v.reshape(BH * S, DH)
    sflat = s_idx_g.reshape(BH * S)
    qks = _gather_rows(qk2, sflat).reshape(BH, S, DH)
    vs = _gather_rows(v2, sflat).reshape(BH, S, DH)

    ts = s_idx_g.reshape(BH, NC, 1, C)
    o_s = _attention(qks, vs, ts, interpret=interpret)

    o_rows = _gather_rows(o_s.reshape(BH * S, DH), u_idx_g.reshape(BH * S))
    o_t = o_rows.reshape(B, H, S, DH).transpose(0, 2, 1, 3).reshape(B, S, D)

    return _stage6(x, o_t, Wo, ln2_g, ln2_b, W1, b1, W2, b2, Wout, bout,
                   interpret=interpret)


def kernel(x, rot, Wqk, Wv, Wo, ln1_g, ln1_b, W1, b1, W2, b2,
           ln2_g, ln2_b, Wout, bout):
    return _pipeline(x, rot, Wqk, Wv, Wo, ln1_g, ln1_b, W1, b1, W2, b2,
                     ln2_g, ln2_b, Wout, bout)


# speedup vs baseline: 3.1867x; 3.1867x over previous
import functools
import math

import jax
import jax.numpy as jnp
from jax import lax
from jax.experimental import pallas as pl
from jax.experimental.pallas import tpu as pltpu

B, S, D, H, C, NB, DFF, V = 2, 4096, 1024, 8, 128, 32, 4096, 256
DH = D // H
NC = S // C
BH = B * H
NBK = 2 * NB


_SB1 = 512


def _stage1_body(x_ref, g_ref, b_ref, wqk_ref, wv_ref, rot_ref,
                 qk_ref, v_ref, bk_ref):
    x = x_ref[0]
    mu = jnp.mean(x, axis=-1, keepdims=True)
    var = jnp.mean((x - mu) * (x - mu), axis=-1, keepdims=True)
    h = (x - mu) * lax.rsqrt(var + 1e-6) * g_ref[...] + b_ref[...]
    qk = jnp.dot(h, wqk_ref[...], preferred_element_type=jnp.float32)
    v = jnp.dot(h, wv_ref[...], preferred_element_type=jnp.float32)
    rx = jnp.dot(qk, rot_ref[...], preferred_element_type=jnp.float32)
    cols = []
    for hh in range(H):
        qk_ref[0, hh] = qk[:, hh * DH:(hh + 1) * DH]
        v_ref[0, hh] = v[:, hh * DH:(hh + 1) * DH]
        r = rx[:, hh * NBK:(hh + 1) * NBK]
        m = jnp.max(r, axis=-1, keepdims=True)
        ii = lax.broadcasted_iota(jnp.int32, r.shape, 1)
        idx = jnp.min(jnp.where(r == m, ii, NBK), axis=-1)
        cols.append(idx[:, None])
    bk_ref[0] = jnp.concatenate(cols, axis=1)


def _stage1(x, ln1_g, ln1_b, Wqk, Wv, rot_bd, interpret=False):
    grid = (B, S // _SB1)
    return pl.pallas_call(
        _stage1_body,
        grid=grid,
        in_specs=[
            pl.BlockSpec((1, _SB1, D), lambda b, i: (b, i, 0)),
            pl.BlockSpec((D,), lambda b, i: (0,)),
            pl.BlockSpec((D,), lambda b, i: (0,)),
            pl.BlockSpec((D, D), lambda b, i: (0, 0)),
            pl.BlockSpec((D, D), lambda b, i: (0, 0)),
            pl.BlockSpec((D, H * NBK), lambda b, i: (0, 0)),
        ],
        out_specs=[
            pl.BlockSpec((1, H, _SB1, DH), lambda b, i: (b, 0, i, 0)),
            pl.BlockSpec((1, H, _SB1, DH), lambda b, i: (b, 0, i, 0)),
            pl.BlockSpec((1, _SB1, H), lambda b, i: (b, i, 0)),
        ],
        out_shape=[
            jax.ShapeDtypeStruct((B, H, S, DH), jnp.float32),
            jax.ShapeDtypeStruct((B, H, S, DH), jnp.float32),
            jax.ShapeDtypeStruct((B, S, H), jnp.int32),
        ],
        interpret=interpret,
    )(x, ln1_g, ln1_b, Wqk, Wv, rot_bd)


def _attn_body(q_ref, kp_ref, kc_ref, vp_ref, vc_ref, tp_ref, tc_ref, o_ref):
    p = pl.program_id(0)
    q = q_ref[0]
    k2 = jnp.concatenate([kp_ref[0], kc_ref[0]], axis=0)
    nrm = jnp.sqrt(jnp.sum(k2 * k2, axis=-1, keepdims=True))
    k2 = k2 / (nrm + 1e-6)
    s = lax.dot_general(q, k2, (((1,), (1,)), ((), ())),
                        preferred_element_type=jnp.float32)
    s = s * (1.0 / math.sqrt(DH))
    base = p * S
    tq = (tc_ref[0, 0] - base).astype(jnp.float32)
    tk = (jnp.concatenate([tp_ref[0, 0], tc_ref[0, 0]], axis=1)
          - base).astype(jnp.float32)
    tqc = jnp.transpose(tq)
    s = jnp.where(tqc < tk, -1e9, s)
    s = jnp.where(tqc == tk, -1e5, s)
    m = jnp.max(s, axis=-1, keepdims=True)
    e = jnp.exp(s - m)
    a = e / jnp.sum(e, axis=-1, keepdims=True)
    v2 = jnp.concatenate([vp_ref[0], vc_ref[0]], axis=0)
    o_ref[0] = jnp.dot(a, v2, preferred_element_type=jnp.float32)


def _attention(qks, vs, ts, interpret=False):
    grid = (BH, NC)
    prev = lambda n: (n + NC - 1) % NC
    return pl.pallas_call(
        _attn_body,
        grid=grid,
        in_specs=[
            pl.BlockSpec((1, C, DH), lambda p, n: (p, n, 0)),
            pl.BlockSpec((1, C, DH), lambda p, n: (p, prev(n), 0)),
            pl.BlockSpec((1, C, DH), lambda p, n: (p, n, 0)),
            pl.BlockSpec((1, C, DH), lambda p, n: (p, prev(n), 0)),
            pl.BlockSpec((1, C, DH), lambda p, n: (p, n, 0)),
            pl.BlockSpec((1, 1, 1, C), lambda p, n: (p, prev(n), 0, 0)),
            pl.BlockSpec((1, 1, 1, C), lambda p, n: (p, n, 0, 0)),
        ],
        out_specs=pl.BlockSpec((1, C, DH), lambda p, n: (p, n, 0)),
        out_shape=jax.ShapeDtypeStruct((BH, S, DH), jnp.float32),
        interpret=interpret,
    )(qks, qks, qks, vs, vs, ts, ts)


_SB6 = 256
_KCH = 4


def _stage6_body(x_ref, o_ref, wo_ref, g2_ref, b2_ref, w1_ref, bb1_ref,
                 w2_ref, bb2_ref, wout_ref, bout_ref, out_ref):
    R = 2 * _SB6
    x2 = x_ref[...].reshape(R, D)
    o2 = o_ref[...].reshape(R, D)
    y1 = x2 + jnp.dot(o2, wo_ref[...], preferred_element_type=jnp.float32)
    mu = jnp.mean(y1, axis=-1, keepdims=True)
    var = jnp.mean((y1 - mu) * (y1 - mu), axis=-1, keepdims=True)
    h2 = (y1 - mu) * lax.rsqrt(var + 1e-6) * g2_ref[...] + b2_ref[...]
    kc = DFF // _KCH
    acc = jnp.zeros((R, D), jnp.float32)
    for j in range(_KCH):
        hj = jnp.dot(h2, w1_ref[:, j * kc:(j + 1) * kc],
                     preferred_element_type=jnp.float32)
        hj = jnp.maximum(hj + bb1_ref[j * kc:(j + 1) * kc], 0.0)
        acc = acc + jnp.dot(hj, w2_ref[j * kc:(j + 1) * kc, :],
                            preferred_element_type=jnp.float32)
    y2 = x2 + acc + bb2_ref[...]
    lg = (jnp.dot(y1, wout_ref[:D], preferred_element_type=jnp.float32)
          + jnp.dot(y2, wout_ref[D:], preferred_element_type=jnp.float32)
          + bout_ref[...])
    l0 = lg[:_SB6]
    l1 = lg[_SB6:]
    m = jnp.maximum(l0, l1)
    e0 = jnp.exp(l0 - m)
    e1 = jnp.exp(l1 - m)
    ssum = e0 + e1
    out_ref[0] = e0 / ssum
    out_ref[1] = e1 / ssum


def _stage6(x, o_t, Wo, ln2_g, ln2_b, W1, b1, W2, b2, Wout, bout,
            interpret=False):
    grid = (S // _SB6,)
    return pl.pallas_call(
        _stage6_body,
        grid=grid,
        in_specs=[
            pl.BlockSpec((B, _SB6, D), lambda i: (0, i, 0)),
            pl.BlockSpec((B, _SB6, D), lambda i: (0, i, 0)),
            pl.BlockSpec((D, D), lambda i: (0, 0)),
            pl.BlockSpec((D,), lambda i: (0,)),
            pl.BlockSpec((D,), lambda i: (0,)),
            pl.BlockSpec((D, DFF), lambda i: (0, 0)),
            pl.BlockSpec((DFF,), lambda i: (0,)),
            pl.BlockSpec((DFF, D), lambda i: (0, 0)),
            pl.BlockSpec((D,), lambda i: (0,)),
            pl.BlockSpec((2 * D, V), lambda i: (0, 0)),
            pl.BlockSpec((V,), lambda i: (0,)),
        ],
        out_specs=pl.BlockSpec((B, _SB6, V), lambda i: (0, i, 0)),
        out_shape=jax.ShapeDtypeStruct((B, S, V), jnp.float32),
        interpret=interpret,
    )(x, o_t, Wo, ln2_g, ln2_b, W1, b1, W2, b2, Wout, bout)


def _sort_pairs(bk_bsh):
    bk = bk_bsh.transpose(0, 2, 1).reshape(BH, S)
    t = jnp.arange(S, dtype=jnp.int32)
    sticker = bk * S + t[None, :]
    s_idx = jnp.argsort(sticker, axis=-1).astype(jnp.int32)
    u_idx = jnp.argsort(s_idx, axis=-1).astype(jnp.int32)
    base = (jnp.arange(BH, dtype=jnp.int32) * S)[:, None]
    return s_idx + base, u_idx + base


def _gather_rows(table, idx_flat):
    return jnp.take(table, idx_flat, axis=0)


def _pipeline(x, rot, Wqk, Wv, Wo, ln1_g, ln1_b, W1, b1, W2, b2,
              ln2_g, ln2_b, Wout, bout, interpret=False):
    rotc = jnp.concatenate([rot, -rot], axis=-1)
    rot_bd = jnp.zeros((D, H * NBK), jnp.float32)
    for hh in range(H):
        rot_bd = rot_bd.at[hh * DH:(hh + 1) * DH,
                           hh * NBK:(hh + 1) * NBK].set(rotc[hh])

    qk, v, bk_sbh = _stage1(x, ln1_g, ln1_b, Wqk, Wv, rot_bd,
                            interpret=interpret)

    s_idx_g, u_idx_g = _sort_pairs(bk_sbh)

    qk2 = qk.reshape(BH * S, DH)
    v2 = v.reshape(BH * S, DH)
    sflat = s_idx_g.reshape(BH * S)
    qks = _gather_rows(qk2, sflat).reshape(BH, S, DH)
    vs = _gather_rows(v2, sflat).reshape(BH, S, DH)

    ts = s_idx_g.reshape(BH, NC, 1, C)
    o_s = _attention(qks, vs, ts, interpret=interpret)

    o_rows = _gather_rows(o_s.reshape(BH * S, DH), u_idx_g.reshape(BH * S))
    o_t = o_rows.reshape(B, H, S, DH).transpose(0, 2, 1, 3).reshape(B, S, D)

    return _stage6(x, o_t, Wo, ln2_g, ln2_b, W1, b1, W2, b2, Wout, bout,
                   interpret=interpret)


def kernel(x, rot, Wqk, Wv, Wo, ln1_g, ln1_b, W1, b1, W2, b2,
           ln2_g, ln2_b, Wout, bout):
    return _pipeline(x, rot, Wqk, Wv, Wo, ln1_g, ln1_b, W1, b1, W2, b2,
                     ln2_g, ln2_b, Wout, bout)
